# Initial kernel scaffold; baseline (speedup 1.0000x reference)
import functools

import jax
import jax.numpy as jnp
from jax import lax
from jax.experimental import pallas as pl
from jax.experimental.pallas import tpu as pltpu
from jax.experimental.pallas import tpu_sc as plsc

N = 10000
E = 320000
NC = 2
NS = 16
NW = NC * NS
EPW = E // NW
CHUNK = 125
NCHUNK = EPW // CHUNK
ACC_ROWS = 10240
ZROWS = 16
ROWS_PER_TILE_OUT = N // NS
OUT_CHUNKS = ROWS_PER_TILE_OUT // CHUNK

_MESH = plsc.VectorSubcoreMesh(core_axis_name="c", subcore_axis_name="s")


def _make_seg_sum(d, with_deg):
    out_type = [jax.ShapeDtypeStruct((NC, N, d), jnp.float32)]
    scratch = [
        pltpu.VMEM((NCHUNK, CHUNK), jnp.int32),
        pltpu.VMEM((NCHUNK, CHUNK), jnp.int32),
        pltpu.VMEM((CHUNK, d), jnp.float32),
        pltpu.VMEM((ZROWS, d), jnp.float32),
        pltpu.VMEM_SHARED((ACC_ROWS, d), jnp.float32),
        pltpu.SemaphoreType.DMA,
    ]
    if with_deg:
        out_type.append(jax.ShapeDtypeStruct((NC, N, 16), jnp.float32))
        scratch += [
            pltpu.VMEM((CHUNK, 16), jnp.float32),
            pltpu.VMEM((ZROWS, 16), jnp.float32),
            pltpu.VMEM((CHUNK, 16), jnp.float32),
            pltpu.VMEM_SHARED((ACC_ROWS, 16), jnp.float32),
        ]

    def body(y_hbm, src_hbm, dst_hbm, *rest):
        if with_deg:
            (out_hbm, deg_hbm, src_v, dst_v, rows_v, zb_v, acc_s, sem,
             ones_v, zd_v, deg_stage_v, dacc_s) = rest
        else:
            out_hbm, src_v, dst_v, rows_v, zb_v, acc_s, sem = rest
        c = lax.axis_index("c")
        s = lax.axis_index("s")
        w = c * NS + s

        for r in range(ZROWS):
            for k in range(d // 16):
                zb_v[r, pl.ds(k * 16, 16)] = jnp.zeros((16,), jnp.float32)
        if with_deg:
            for r in range(ZROWS):
                zd_v[r, :] = jnp.zeros((16,), jnp.float32)
            for r in range(CHUNK):
                ones_v[r, :] = jnp.ones((16,), jnp.float32)

        rows_per_tile = ACC_ROWS // NS
        zbase = s * rows_per_tile

        def zloop(j, carry):
            pltpu.sync_copy(zb_v, acc_s.at[pl.ds(zbase + j * ZROWS, ZROWS)])
            if with_deg:
                pltpu.sync_copy(zd_v, dacc_s.at[pl.ds(zbase + j * ZROWS, ZROWS)])
            return carry

        lax.fori_loop(0, rows_per_tile // ZROWS, zloop, 0)
        plsc.subcore_barrier()

        pltpu.sync_copy(src_hbm.at[w], src_v)
        pltpu.sync_copy(dst_hbm.at[w], dst_v)

        def eloop(j, carry):
            pltpu.async_copy(y_hbm.at[src_v.at[j]], rows_v, sem).wait()
            pltpu.sync_copy(rows_v, acc_s.at[dst_v.at[j]], add=True)
            if with_deg:
                pltpu.sync_copy(ones_v, dacc_s.at[dst_v.at[j]], add=True)
            return carry

        lax.fori_loop(0, NCHUNK, eloop, 0)
        plsc.subcore_barrier()

        obase = s * ROWS_PER_TILE_OUT

        def oloop(j, carry):
            pltpu.sync_copy(acc_s.at[pl.ds(obase + j * CHUNK, CHUNK)], rows_v)
            pltpu.sync_copy(rows_v, out_hbm.at[c, pl.ds(obase + j * CHUNK, CHUNK)])
            if with_deg:
                pltpu.sync_copy(dacc_s.at[pl.ds(obase + j * CHUNK, CHUNK)],
                                deg_stage_v)
                pltpu.sync_copy(deg_stage_v,
                                deg_hbm.at[c, pl.ds(obase + j * CHUNK, CHUNK)])
            return carry

        lax.fori_loop(0, OUT_CHUNKS, oloop, 0)

    return pl.kernel(body, out_type=out_type, mesh=_MESH, scratch_types=scratch)


_seg_sum_128_deg = _make_seg_sum(128, True)
_seg_sum_128 = _make_seg_sum(128, False)
_seg_sum_32 = _make_seg_sum(32, False)

_DN = (((1,), (1,)), ((), ()))


def _dot(a, b):
    return lax.dot_general(a, b, _DN, precision=lax.Precision.HIGHEST,
                           preferred_element_type=jnp.float32)


def _tc_first(x_ref, wl_ref, wr_ref, b_ref, yl_ref, yr_ref):
    x = x_ref[...]
    yl_ref[...] = _dot(x, wl_ref[...])
    yr_ref[...] = _dot(x, wr_ref[...]) + b_ref[...]


def _tc_mid(s_ref, deg_ref, yr_ref, wl_ref, wr_ref, b_ref, yl_o, yr_o):
    deg = jnp.maximum(deg_ref[0, :, 0:1] + deg_ref[1, :, 0:1], 1.0)
    agg = (s_ref[0] + s_ref[1]) / deg
    h = jnp.maximum(agg + yr_ref[...], 0.0)
    yl_o[...] = _dot(h, wl_ref[...])
    yr_o[...] = _dot(h, wr_ref[...]) + b_ref[...]


def _tc_final(s_ref, deg_ref, yr_ref, w4_ref, b4_ref, out_ref):
    deg = jnp.maximum(deg_ref[0, :, 0:1] + deg_ref[1, :, 0:1], 1.0)
    agg = (s_ref[0] + s_ref[1]) / deg
    h = jnp.maximum(agg + yr_ref[...], 0.0)
    out_ref[...] = _dot(h, w4_ref[...]) + b4_ref[...]


def _f32(shape):
    return jax.ShapeDtypeStruct(shape, jnp.float32)


def kernel(x, edge_index, Wl1, Wr1, b1, Wl2, Wr2, b2, Wl3, Wr3, b3, W4, b4):
    src3 = edge_index[0].astype(jnp.int32).reshape(NW, NCHUNK, CHUNK)
    dst3 = edge_index[1].astype(jnp.int32).reshape(NW, NCHUNK, CHUNK)

    yl1, yr1 = pl.pallas_call(
        _tc_first, out_shape=[_f32((N, 128)), _f32((N, 128))])(
        x, Wl1, Wr1, b1.reshape(1, 128))
    s1, deg = _seg_sum_128_deg(yl1, src3, dst3)

    yl2, yr2 = pl.pallas_call(
        _tc_mid, out_shape=[_f32((N, 128)), _f32((N, 128))])(
        s1, deg, yr1, Wl2, Wr2, b2.reshape(1, 128))
    s2 = _seg_sum_128(yl2, src3, dst3)

    yl3, yr3 = pl.pallas_call(
        _tc_mid, out_shape=[_f32((N, 32)), _f32((N, 32))])(
        s2, deg, yr2, Wl3, Wr3, b3.reshape(1, 32))
    s3 = _seg_sum_32(yl3, src3, dst3)

    out = pl.pallas_call(
        _tc_final, out_shape=_f32((N, 1)))(
        s3, deg, yr3, W4, b4.reshape(1, 1))
    return out


# speedup vs baseline: 7.3076x; 7.3076x over previous
import functools

import jax
import jax.numpy as jnp
from jax import lax
from jax.experimental import pallas as pl
from jax.experimental.pallas import tpu as pltpu
from jax.experimental.pallas import tpu_sc as plsc

N = 10000
E = 320000
NC = 2
NS = 16
NW = NC * NS
EPW = E // NW
CHUNK = 125
NCHUNK = EPW // CHUNK
GCHUNK = 8
GROUPS = NCHUNK // GCHUNK
ACC_ROWS = 10240
ZROWS = 16
ROWS_PER_TILE_OUT = ACC_ROWS // NS
OUT_CHUNK = 128
OUT_CHUNKS = ROWS_PER_TILE_OUT // OUT_CHUNK

_MESH = plsc.VectorSubcoreMesh(core_axis_name="c", subcore_axis_name="s")


def _make_seg_sum(d):
    out_type = [jax.ShapeDtypeStruct((NC, ACC_ROWS, d), jnp.float32)]
    scratch = [
        pltpu.VMEM((GCHUNK, CHUNK), jnp.int32),
        pltpu.VMEM((GCHUNK, CHUNK), jnp.int32),
        pltpu.VMEM((CHUNK, d), jnp.float32),
        pltpu.VMEM_SHARED((ACC_ROWS, d), jnp.float32),
        pltpu.SemaphoreType.DMA,
    ]

    def body(y_hbm, src_hbm, dst_hbm, out_hbm, src_v, dst_v, stage_v, acc_s,
             sem):
        c = lax.axis_index("c")
        s = lax.axis_index("s")
        w = c * NS + s

        for r in range(ZROWS):
            for k in range(d // 16):
                stage_v[r, pl.ds(k * 16, 16)] = jnp.zeros((16,), jnp.float32)

        rows_per_tile = ACC_ROWS // NS
        zbase = s * rows_per_tile

        def zloop(j, carry):
            pltpu.sync_copy(stage_v.at[pl.ds(0, ZROWS)],
                            acc_s.at[pl.ds(zbase + j * ZROWS, ZROWS)])
            return carry

        lax.fori_loop(0, rows_per_tile // ZROWS, zloop, 0)
        plsc.subcore_barrier()

        def gloop(g, carry):
            pltpu.sync_copy(src_hbm.at[w, g], src_v)
            pltpu.sync_copy(dst_hbm.at[w, g], dst_v)
            for j in range(GCHUNK):
                pltpu.async_copy(y_hbm.at[src_v.at[j]], stage_v, sem).wait()
                pltpu.sync_copy(stage_v, acc_s.at[dst_v.at[j]], add=True)
            return carry

        lax.fori_loop(0, GROUPS, gloop, 0)
        plsc.subcore_barrier()

        obase = s * ROWS_PER_TILE_OUT
        sl = pl.ds(obase, ROWS_PER_TILE_OUT)
        pltpu.sync_copy(acc_s.at[sl], out_hbm.at[c, sl])

    return pl.kernel(body, out_type=out_type, mesh=_MESH, scratch_types=scratch)


def _make_deg():
    out_type = [jax.ShapeDtypeStruct((NC, ACC_ROWS, 128), jnp.float32)]
    scratch = [
        pltpu.VMEM((GCHUNK, CHUNK), jnp.int32),
        pltpu.VMEM((CHUNK, 128), jnp.float32),
        pltpu.VMEM_SHARED((ACC_ROWS, 128), jnp.float32),
        pltpu.SemaphoreType.DMA,
    ]

    def body(dst_hbm, deg_hbm, dst_v, ones_v, dacc_s, sem):
        c = lax.axis_index("c")
        s = lax.axis_index("s")
        w = c * NS + s

        for r in range(ZROWS):
            for k in range(8):
                ones_v[r, pl.ds(k * 16, 16)] = jnp.zeros((16,), jnp.float32)

        rows_per_tile = ACC_ROWS // NS
        zbase = s * rows_per_tile

        def zloop(j, carry):
            pltpu.sync_copy(ones_v.at[pl.ds(0, ZROWS)],
                            dacc_s.at[pl.ds(zbase + j * ZROWS, ZROWS)])
            return carry

        lax.fori_loop(0, rows_per_tile // ZROWS, zloop, 0)

        for r in range(CHUNK):
            for k in range(8):
                ones_v[r, pl.ds(k * 16, 16)] = jnp.ones((16,), jnp.float32)
        plsc.subcore_barrier()

        def gloop(g, carry):
            pltpu.sync_copy(dst_hbm.at[w, g], dst_v)
            for j in range(GCHUNK):
                pltpu.sync_copy(ones_v, dacc_s.at[dst_v.at[j]], add=True)
            return carry

        lax.fori_loop(0, GROUPS, gloop, 0)
        plsc.subcore_barrier()

        obase = s * ROWS_PER_TILE_OUT
        sl = pl.ds(obase, ROWS_PER_TILE_OUT)
        pltpu.sync_copy(dacc_s.at[sl], deg_hbm.at[c, sl])

    return pl.kernel(body, out_type=out_type, mesh=_MESH, scratch_types=scratch)


_seg_sum_128 = _make_seg_sum(128)
_deg_count = _make_deg()

_DN = (((1,), (1,)), ((), ()))


def _dot(a, b):
    return lax.dot_general(a, b, _DN, precision=lax.Precision.HIGHEST,
                           preferred_element_type=jnp.float32)


def _tc_first(x_ref, wl_ref, wr_ref, b_ref, yl_ref, yr_ref):
    x = x_ref[...]
    yl_ref[...] = _dot(x, wl_ref[...])
    yr_ref[...] = _dot(x, wr_ref[...]) + b_ref[...]


def _tc_mid(s_ref, deg_ref, yr_ref, wl_ref, wr_ref, b_ref, yl_o, yr_o):
    deg = jnp.maximum(deg_ref[0, 0:N, 0:1] + deg_ref[1, 0:N, 0:1], 1.0)
    agg = (s_ref[0, 0:N] + s_ref[1, 0:N]) / deg
    h = jnp.maximum(agg + yr_ref[...], 0.0)
    yl_o[...] = _dot(h, wl_ref[...])
    yr_o[...] = _dot(h, wr_ref[...]) + b_ref[...]


def _tc_mid_b(s_ref, deg_ref, yr_ref, wr_ref, b_ref, h_o, yr_o):
    deg = jnp.maximum(deg_ref[0, 0:N, 0:1] + deg_ref[1, 0:N, 0:1], 1.0)
    agg = (s_ref[0, 0:N] + s_ref[1, 0:N]) / deg
    h = jnp.maximum(agg + yr_ref[...], 0.0)
    h_o[...] = h
    yr_o[...] = _dot(h, wr_ref[...]) + b_ref[...]


def _tc_final(s_ref, deg_ref, yr_ref, wl_ref, w4_ref, b4_ref, out_ref):
    deg = jnp.maximum(deg_ref[0, 0:N, 0:1] + deg_ref[1, 0:N, 0:1], 1.0)
    agg = (s_ref[0, 0:N] + s_ref[1, 0:N]) / deg
    h = jnp.maximum(_dot(agg, wl_ref[...]) + yr_ref[...], 0.0)
    out_ref[...] = (jnp.sum(h * w4_ref[...], axis=1, keepdims=True)
                    + b4_ref[0, 0])


def _f32(shape):
    return jax.ShapeDtypeStruct(shape, jnp.float32)


def kernel(x, edge_index, Wl1, Wr1, b1, Wl2, Wr2, b2, Wl3, Wr3, b3, W4, b4):
    src3 = edge_index[0].astype(jnp.int32).reshape(NW, GROUPS, GCHUNK, CHUNK)
    dst3 = edge_index[1].astype(jnp.int32).reshape(NW, GROUPS, GCHUNK, CHUNK)

    yl1, yr1 = pl.pallas_call(
        _tc_first, out_shape=[_f32((N, 128)), _f32((N, 128))])(
        x, Wl1, Wr1, b1.reshape(1, 128))
    (deg,) = _deg_count(dst3)
    (s1,) = _seg_sum_128(yl1, src3, dst3)

    yl2, yr2 = pl.pallas_call(
        _tc_mid, out_shape=[_f32((N, 128)), _f32((N, 128))])(
        s1, deg, yr1, Wl2, Wr2, b2.reshape(1, 128))
    (s2,) = _seg_sum_128(yl2, src3, dst3)

    h2, yr3 = pl.pallas_call(
        _tc_mid_b, out_shape=[_f32((N, 128)), _f32((N, 32))])(
        s2, deg, yr2, Wr3, b3.reshape(1, 32))
    (s3,) = _seg_sum_128(h2, src3, dst3)

    out = pl.pallas_call(
        _tc_final, out_shape=_f32((N, 1)))(
        s3, deg, yr3, Wl3, W4, b4.reshape(1, 1))
    return out


# speedup vs baseline: 10.7008x; 1.4643x over previous
import functools

import jax
import jax.numpy as jnp
from jax import lax
from jax.experimental import pallas as pl
from jax.experimental.pallas import tpu as pltpu
from jax.experimental.pallas import tpu_sc as plsc

N = 10000
E = 320000
NC = 2
NS = 16
NW = NC * NS
EPW = E // NW
CHUNK = 100
IG = 20
NG = EPW // (IG * CHUNK)
ACC_ROWS = 10112
RPT = ACC_ROWS // NS
ZC = 96

_MESH = plsc.VectorSubcoreMesh(core_axis_name="c", subcore_axis_name="s")


def _zero_rows(buf, nrows, d):
    def fill(r, carry):
        for k in range(d // 16):
            buf[r, pl.ds(k * 16, 16)] = jnp.zeros((16,), jnp.float32)
        return carry
    lax.fori_loop(0, nrows, fill, 0)


def _zero_acc_slice(acc_s, src, s):
    zbase = s * RPT
    nfull = RPT // ZC
    rem = RPT - nfull * ZC

    def zloop(j, carry):
        pltpu.sync_copy(src.at[pl.ds(0, ZC)],
                        acc_s.at[pl.ds(zbase + j * ZC, ZC)])
        return carry

    lax.fori_loop(0, nfull, zloop, 0)
    pltpu.sync_copy(src.at[pl.ds(0, rem)],
                    acc_s.at[pl.ds(zbase + nfull * ZC, rem)])


def _make_seg_sum(d):
    out_type = [jax.ShapeDtypeStruct((NC, ACC_ROWS, d), jnp.float32)]
    scratch = [
        pltpu.VMEM((IG, CHUNK), jnp.int32),
        pltpu.VMEM((IG, CHUNK), jnp.int32),
        pltpu.VMEM((2, CHUNK, d), jnp.float32),
        pltpu.VMEM_SHARED((ACC_ROWS, d), jnp.float32),
        pltpu.SemaphoreType.DMA,
        pltpu.SemaphoreType.DMA,
        pltpu.SemaphoreType.DMA,
        pltpu.SemaphoreType.DMA,
    ]

    def body(y_hbm, src_hbm, dst_hbm, out_hbm, src_v, dst_v, rows_v, acc_s,
             gsem0, gsem1, ssem0, ssem1):
        c = lax.axis_index("c")
        s = lax.axis_index("s")
        w = c * NS + s
        gsems = (gsem0, gsem1)
        ssems = (ssem0, ssem1)

        _zero_rows(rows_v.at[0], ZC, d)
        _zero_acc_slice(acc_s, rows_v.at[0], s)
        plsc.subcore_barrier()

        def gloop(g, carry):
            pltpu.sync_copy(src_hbm.at[w, g], src_v)
            pltpu.sync_copy(dst_hbm.at[w, g], dst_v)
            cpg = [None, None]
            cps = [None, None]
            cpg[0] = pltpu.async_copy(y_hbm.at[src_v.at[0]], rows_v.at[0],
                                      gsems[0])
            for j in range(IG):
                b = j & 1
                if j + 1 < IG:
                    if cps[1 - b] is not None:
                        cps[1 - b].wait()
                    cpg[1 - b] = pltpu.async_copy(
                        y_hbm.at[src_v.at[j + 1]], rows_v.at[1 - b],
                        gsems[1 - b])
                cpg[b].wait()
                cps[b] = pltpu.async_copy(rows_v.at[b], acc_s.at[dst_v.at[j]],
                                          ssems[b], add=True)
            cps[0].wait()
            cps[1].wait()
            return carry

        lax.fori_loop(0, NG, gloop, 0)
        plsc.subcore_barrier()

        sl = pl.ds(s * RPT, RPT)
        pltpu.sync_copy(acc_s.at[sl], out_hbm.at[c, sl])

    return pl.kernel(body, out_type=out_type, mesh=_MESH, scratch_types=scratch)


def _make_deg():
    out_type = [jax.ShapeDtypeStruct((NC, ACC_ROWS, 128), jnp.float32)]
    scratch = [
        pltpu.VMEM((IG, CHUNK), jnp.int32),
        pltpu.VMEM((CHUNK, 128), jnp.float32),
        pltpu.VMEM_SHARED((ACC_ROWS, 128), jnp.float32),
        pltpu.SemaphoreType.DMA,
    ]

    def body(dst_hbm, deg_hbm, dst_v, ones_v, dacc_s, sem):
        c = lax.axis_index("c")
        s = lax.axis_index("s")
        w = c * NS + s

        _zero_rows(ones_v, ZC, 128)
        _zero_acc_slice(dacc_s, ones_v, s)

        def fill(r, carry):
            for k in range(8):
                ones_v[r, pl.ds(k * 16, 16)] = jnp.ones((16,), jnp.float32)
            return carry
        lax.fori_loop(0, CHUNK, fill, 0)
        plsc.subcore_barrier()

        def gloop(g, carry):
            pltpu.sync_copy(dst_hbm.at[w, g], dst_v)
            cps = []
            for j in range(IG):
                cps.append(pltpu.async_copy(ones_v, dacc_s.at[dst_v.at[j]],
                                            sem, add=True))
            for cp in cps:
                cp.wait()
            return carry

        lax.fori_loop(0, NG, gloop, 0)
        plsc.subcore_barrier()

        sl = pl.ds(s * RPT, RPT)
        pltpu.sync_copy(dacc_s.at[sl], deg_hbm.at[c, sl])

    return pl.kernel(body, out_type=out_type, mesh=_MESH, scratch_types=scratch)


_seg_sum_128 = _make_seg_sum(128)
_deg_count = _make_deg()

_DN = (((1,), (1,)), ((), ()))


def _dot(a, b):
    return lax.dot_general(a, b, _DN, preferred_element_type=jnp.float32)


def _agg(s_ref, deg_ref):
    deg = jnp.maximum(deg_ref[0, 0:N, 0:1] + deg_ref[1, 0:N, 0:1], 1.0)
    return (s_ref[0, 0:N] + s_ref[1, 0:N]) / deg


def _tc_layer(s_ref, deg_ref, h_ref, wl_ref, wr_ref, b_ref, h_o):
    agg = _agg(s_ref, deg_ref)
    h = h_ref[...]
    h_o[...] = jnp.maximum(
        _dot(agg, wl_ref[...]) + _dot(h, wr_ref[...]) + b_ref[...], 0.0)


def _tc_final(s_ref, deg_ref, h_ref, wl_ref, wr_ref, b_ref, w4_ref, b4_ref,
              out_ref):
    agg = _agg(s_ref, deg_ref)
    h3 = jnp.maximum(
        _dot(agg, wl_ref[...]) + _dot(h_ref[...], wr_ref[...]) + b_ref[...],
        0.0)
    out_ref[...] = (jnp.sum(h3 * w4_ref[...], axis=1, keepdims=True)
                    + b4_ref[0, 0])


def _f32(shape):
    return jax.ShapeDtypeStruct(shape, jnp.float32)


def kernel(x, edge_index, Wl1, Wr1, b1, Wl2, Wr2, b2, Wl3, Wr3, b3, W4, b4):
    src3 = edge_index[0].astype(jnp.int32).reshape(NW, NG, IG, CHUNK)
    dst3 = edge_index[1].astype(jnp.int32).reshape(NW, NG, IG, CHUNK)

    (deg,) = _deg_count(dst3)
    (s1,) = _seg_sum_128(x, src3, dst3)
    h1 = pl.pallas_call(_tc_layer, out_shape=_f32((N, 128)))(
        s1, deg, x, Wl1, Wr1, b1.reshape(1, 128))

    (s2,) = _seg_sum_128(h1, src3, dst3)
    h2 = pl.pallas_call(_tc_layer, out_shape=_f32((N, 128)))(
        s2, deg, h1, Wl2, Wr2, b2.reshape(1, 128))

    (s3,) = _seg_sum_128(h2, src3, dst3)
    out = pl.pallas_call(_tc_final, out_shape=_f32((N, 1)))(
        s3, deg, h2, Wl3, Wr3, b3.reshape(1, 32), W4, b4.reshape(1, 1))
    return out


# speedup vs baseline: 11.0343x; 1.0312x over previous
import functools

import jax
import jax.numpy as jnp
from jax import lax
from jax.experimental import pallas as pl
from jax.experimental.pallas import tpu as pltpu
from jax.experimental.pallas import tpu_sc as plsc

N = 10000
E = 320000
NC = 2
NS = 16
NW = NC * NS
EPW = E // NW
CHUNK = 100
IG = 20
NG = EPW // (IG * CHUNK)
ACC_ROWS = 10112
RPT = ACC_ROWS // NS
ZC = 96

_MESH = plsc.VectorSubcoreMesh(core_axis_name="c", subcore_axis_name="s")


def _zero_rows(buf, nrows, d):
    def fill(r, carry):
        for k in range(d // 16):
            buf[r, pl.ds(k * 16, 16)] = jnp.zeros((16,), jnp.float32)
        return carry
    lax.fori_loop(0, nrows, fill, 0)


def _zero_acc_slice(acc_s, src, s, zsem):
    zbase = s * RPT
    nfull = RPT // ZC
    rem = RPT - nfull * ZC
    cps = []
    for j in range(nfull):
        cps.append(pltpu.async_copy(
            src.at[pl.ds(0, ZC)], acc_s.at[pl.ds(zbase + j * ZC, ZC)], zsem))
    cps.append(pltpu.async_copy(
        src.at[pl.ds(0, rem)], acc_s.at[pl.ds(zbase + nfull * ZC, rem)], zsem))
    for cp in cps:
        cp.wait()


def _make_seg_sum(d):
    out_type = [jax.ShapeDtypeStruct((NC, ACC_ROWS, d), jnp.float32)]
    scratch = [
        pltpu.VMEM((IG, CHUNK), jnp.int32),
        pltpu.VMEM((IG, CHUNK), jnp.int32),
        pltpu.VMEM((2, CHUNK, d), jnp.float32),
        pltpu.VMEM_SHARED((ACC_ROWS, d), jnp.float32),
        pltpu.SemaphoreType.DMA,
        pltpu.SemaphoreType.DMA,
        pltpu.SemaphoreType.DMA,
        pltpu.SemaphoreType.DMA,
        pltpu.SemaphoreType.DMA,
        pltpu.SemaphoreType.DMA,
    ]

    def body(y_hbm, src_hbm, dst_hbm, out_hbm, src_v, dst_v, rows_v, acc_s,
             gsem0, gsem1, ssem0, ssem1, isem_s, isem_d):
        c = lax.axis_index("c")
        s = lax.axis_index("s")
        w = c * NS + s
        gsems = (gsem0, gsem1)
        ssems = (ssem0, ssem1)

        pltpu.async_copy(src_hbm.at[w, 0], src_v, isem_s)
        pltpu.async_copy(dst_hbm.at[w, 0], dst_v, isem_d)
        _zero_rows(rows_v.at[0], ZC, d)
        _zero_acc_slice(acc_s, rows_v.at[0], s, gsem0)
        plsc.subcore_barrier()

        def gloop(g, carry):
            pltpu.make_async_copy(src_hbm.at[w, g], src_v, isem_s).wait()
            pltpu.make_async_copy(dst_hbm.at[w, g], dst_v, isem_d).wait()
            cpg = [None, None]
            cps = [None, None]
            cpg[0] = pltpu.async_copy(y_hbm.at[src_v.at[0]], rows_v.at[0],
                                      gsems[0])
            for j in range(IG):
                b = j & 1
                if j + 1 < IG:
                    if cps[1 - b] is not None:
                        cps[1 - b].wait()
                    cpg[1 - b] = pltpu.async_copy(
                        y_hbm.at[src_v.at[j + 1]], rows_v.at[1 - b],
                        gsems[1 - b])
                cpg[b].wait()
                cps[b] = pltpu.async_copy(rows_v.at[b], acc_s.at[dst_v.at[j]],
                                          ssems[b], add=True)
            cps[0].wait()
            cps[1].wait()

            @pl.when(g < NG - 1)
            def _prefetch():
                pltpu.async_copy(src_hbm.at[w, g + 1], src_v, isem_s)
                pltpu.async_copy(dst_hbm.at[w, g + 1], dst_v, isem_d)

            return carry

        lax.fori_loop(0, NG, gloop, 0)
        plsc.subcore_barrier()

        sl = pl.ds(s * RPT, RPT)
        pltpu.sync_copy(acc_s.at[sl], out_hbm.at[c, sl])

    return pl.kernel(body, out_type=out_type, mesh=_MESH, scratch_types=scratch)


def _make_deg():
    out_type = [jax.ShapeDtypeStruct((NC, ACC_ROWS, 128), jnp.float32)]
    scratch = [
        pltpu.VMEM((IG, CHUNK), jnp.int32),
        pltpu.VMEM((CHUNK, 128), jnp.float32),
        pltpu.VMEM_SHARED((ACC_ROWS, 128), jnp.float32),
        pltpu.SemaphoreType.DMA,
        pltpu.SemaphoreType.DMA,
    ]

    def body(dst_hbm, deg_hbm, dst_v, ones_v, dacc_s, sem, isem):
        c = lax.axis_index("c")
        s = lax.axis_index("s")
        w = c * NS + s

        pltpu.async_copy(dst_hbm.at[w, 0], dst_v, isem)
        _zero_rows(ones_v, ZC, 128)
        _zero_acc_slice(dacc_s, ones_v, s, sem)

        def fill(r, carry):
            for k in range(8):
                ones_v[r, pl.ds(k * 16, 16)] = jnp.ones((16,), jnp.float32)
            return carry
        lax.fori_loop(0, CHUNK, fill, 0)
        plsc.subcore_barrier()

        def gloop(g, carry):
            pltpu.make_async_copy(dst_hbm.at[w, g], dst_v, isem).wait()
            cps = []
            for j in range(IG):
                cps.append(pltpu.async_copy(ones_v, dacc_s.at[dst_v.at[j]],
                                            sem, add=True))
            for cp in cps:
                cp.wait()

            @pl.when(g < NG - 1)
            def _prefetch():
                pltpu.async_copy(dst_hbm.at[w, g + 1], dst_v, isem)

            return carry

        lax.fori_loop(0, NG, gloop, 0)
        plsc.subcore_barrier()

        sl = pl.ds(s * RPT, RPT)
        pltpu.sync_copy(dacc_s.at[sl], deg_hbm.at[c, sl])

    return pl.kernel(body, out_type=out_type, mesh=_MESH, scratch_types=scratch)


_seg_sum_128 = _make_seg_sum(128)
_deg_count = _make_deg()

_DN = (((1,), (1,)), ((), ()))


def _dot(a, b):
    return lax.dot_general(a, b, _DN, preferred_element_type=jnp.float32)


def _agg(s_ref, deg_ref):
    deg = jnp.maximum(deg_ref[0, 0:N, 0:1] + deg_ref[1, 0:N, 0:1], 1.0)
    return (s_ref[0, 0:N] + s_ref[1, 0:N]) / deg


def _tc_layer(s_ref, deg_ref, h_ref, wl_ref, wr_ref, b_ref, h_o):
    agg = _agg(s_ref, deg_ref)
    h = h_ref[...]
    h_o[...] = jnp.maximum(
        _dot(agg, wl_ref[...]) + _dot(h, wr_ref[...]) + b_ref[...], 0.0)


def _tc_final(s_ref, deg_ref, h_ref, wl_ref, wr_ref, b_ref, w4_ref, b4_ref,
              out_ref):
    agg = _agg(s_ref, deg_ref)
    h3 = jnp.maximum(
        _dot(agg, wl_ref[...]) + _dot(h_ref[...], wr_ref[...]) + b_ref[...],
        0.0)
    out_ref[...] = (jnp.sum(h3 * w4_ref[...], axis=1, keepdims=True)
                    + b4_ref[0, 0])


def _f32(shape):
    return jax.ShapeDtypeStruct(shape, jnp.float32)


def kernel(x, edge_index, Wl1, Wr1, b1, Wl2, Wr2, b2, Wl3, Wr3, b3, W4, b4):
    src3 = edge_index[0].astype(jnp.int32).reshape(NW, NG, IG, CHUNK)
    dst3 = edge_index[1].astype(jnp.int32).reshape(NW, NG, IG, CHUNK)

    (deg,) = _deg_count(dst3)
    (s1,) = _seg_sum_128(x, src3, dst3)
    h1 = pl.pallas_call(_tc_layer, out_shape=_f32((N, 128)))(
        s1, deg, x, Wl1, Wr1, b1.reshape(1, 128))

    (s2,) = _seg_sum_128(h1, src3, dst3)
    h2 = pl.pallas_call(_tc_layer, out_shape=_f32((N, 128)))(
        s2, deg, h1, Wl2, Wr2, b2.reshape(1, 128))

    (s3,) = _seg_sum_128(h2, src3, dst3)
    out = pl.pallas_call(_tc_final, out_shape=_f32((N, 1)))(
        s3, deg, h2, Wl3, Wr3, b3.reshape(1, 32), W4, b4.reshape(1, 1))
    return out


# speedup vs baseline: 11.1412x; 1.0097x over previous
import functools

import jax
import jax.numpy as jnp
from jax import lax
from jax.experimental import pallas as pl
from jax.experimental.pallas import tpu as pltpu
from jax.experimental.pallas import tpu_sc as plsc

N = 10000
E = 320000
NC = 2
NS = 16
NW = NC * NS
EPW = E // NW
CHUNK = 100
IG = 20
NG = EPW // (IG * CHUNK)
ACC_ROWS = 10112
RPT = ACC_ROWS // NS
ZC = 96

_MESH = plsc.VectorSubcoreMesh(core_axis_name="c", subcore_axis_name="s")


def _zero_rows(buf, nrows, d):
    def fill(r, carry):
        for k in range(d // 16):
            buf[r, pl.ds(k * 16, 16)] = jnp.zeros((16,), jnp.float32)
        return carry
    lax.fori_loop(0, nrows, fill, 0)


def _zero_acc_slice(acc_s, src, s, zsem):
    zbase = s * RPT
    nfull = RPT // ZC
    rem = RPT - nfull * ZC
    cps = []
    for j in range(nfull):
        cps.append(pltpu.async_copy(
            src.at[pl.ds(0, ZC)], acc_s.at[pl.ds(zbase + j * ZC, ZC)], zsem))
    cps.append(pltpu.async_copy(
        src.at[pl.ds(0, rem)], acc_s.at[pl.ds(zbase + nfull * ZC, rem)], zsem))
    for cp in cps:
        cp.wait()


def _make_seg_sum(d):
    out_type = [jax.ShapeDtypeStruct((NC, ACC_ROWS, d), jnp.float32)]
    scratch = [
        pltpu.VMEM((IG, CHUNK), jnp.int32),
        pltpu.VMEM((IG, CHUNK), jnp.int32),
        pltpu.VMEM((2, CHUNK, d), jnp.float32),
        pltpu.VMEM_SHARED((ACC_ROWS, d), jnp.float32),
        pltpu.SemaphoreType.DMA,
        pltpu.SemaphoreType.DMA,
        pltpu.SemaphoreType.DMA,
        pltpu.SemaphoreType.DMA,
        pltpu.SemaphoreType.DMA,
        pltpu.SemaphoreType.DMA,
    ]

    def body(y_hbm, src_hbm, dst_hbm, out_hbm, src_v, dst_v, rows_v, acc_s,
             gsem0, gsem1, ssem0, ssem1, isem_s, isem_d):
        c = lax.axis_index("c")
        s = lax.axis_index("s")
        w = c * NS + s
        gsems = (gsem0, gsem1)
        ssems = (ssem0, ssem1)

        pltpu.async_copy(src_hbm.at[w, 0], src_v, isem_s)
        pltpu.async_copy(dst_hbm.at[w, 0], dst_v, isem_d)
        _zero_rows(rows_v.at[0], ZC, d)
        _zero_acc_slice(acc_s, rows_v.at[0], s, gsem0)
        plsc.subcore_barrier()

        def gloop(g, carry):
            pltpu.make_async_copy(src_hbm.at[w, g], src_v, isem_s).wait()
            pltpu.make_async_copy(dst_hbm.at[w, g], dst_v, isem_d).wait()
            cpg = [None, None]
            cps = [None, None]
            cpg[0] = pltpu.async_copy(y_hbm.at[src_v.at[0]], rows_v.at[0],
                                      gsems[0])
            for j in range(IG):
                b = j & 1
                if j + 1 < IG:
                    if cps[1 - b] is not None:
                        cps[1 - b].wait()
                    cpg[1 - b] = pltpu.async_copy(
                        y_hbm.at[src_v.at[j + 1]], rows_v.at[1 - b],
                        gsems[1 - b])
                cpg[b].wait()
                cps[b] = pltpu.async_copy(rows_v.at[b], acc_s.at[dst_v.at[j]],
                                          ssems[b], add=True)
            cps[0].wait()
            cps[1].wait()

            @pl.when(g < NG - 1)
            def _prefetch():
                pltpu.async_copy(src_hbm.at[w, g + 1], src_v, isem_s)
                pltpu.async_copy(dst_hbm.at[w, g + 1], dst_v, isem_d)

            return carry

        lax.fori_loop(0, NG, gloop, 0)
        plsc.subcore_barrier()

        sl = pl.ds(s * RPT, RPT)
        pltpu.sync_copy(acc_s.at[sl], out_hbm.at[c, sl])

    return pl.kernel(body, out_type=out_type, mesh=_MESH, scratch_types=scratch)


def _make_deg():
    out_type = [jax.ShapeDtypeStruct((NC, ACC_ROWS, 128), jnp.float32)]
    scratch = [
        pltpu.VMEM((IG, CHUNK), jnp.int32),
        pltpu.VMEM((CHUNK, 128), jnp.float32),
        pltpu.VMEM_SHARED((ACC_ROWS, 128), jnp.float32),
        pltpu.SemaphoreType.DMA,
        pltpu.SemaphoreType.DMA,
    ]

    def body(dst_hbm, deg_hbm, dst_v, ones_v, dacc_s, sem, isem):
        c = lax.axis_index("c")
        s = lax.axis_index("s")
        w = c * NS + s

        pltpu.async_copy(dst_hbm.at[w, 0], dst_v, isem)
        _zero_rows(ones_v, ZC, 128)
        _zero_acc_slice(dacc_s, ones_v, s, sem)

        def fill(r, carry):
            for k in range(8):
                ones_v[r, pl.ds(k * 16, 16)] = jnp.ones((16,), jnp.float32)
            return carry
        lax.fori_loop(0, CHUNK, fill, 0)
        plsc.subcore_barrier()

        def gloop(g, carry):
            pltpu.make_async_copy(dst_hbm.at[w, g], dst_v, isem).wait()
            cps = []
            for j in range(IG):
                cps.append(pltpu.async_copy(ones_v, dacc_s.at[dst_v.at[j]],
                                            sem, add=True))
            for cp in cps:
                cp.wait()

            @pl.when(g < NG - 1)
            def _prefetch():
                pltpu.async_copy(dst_hbm.at[w, g + 1], dst_v, isem)

            return carry

        lax.fori_loop(0, NG, gloop, 0)
        plsc.subcore_barrier()

        sl = pl.ds(s * RPT, RPT)
        pltpu.sync_copy(dacc_s.at[sl], deg_hbm.at[c, sl])

    return pl.kernel(body, out_type=out_type, mesh=_MESH, scratch_types=scratch)


_seg_sum_128 = _make_seg_sum(128)
_deg_count = _make_deg()

_DN = (((1,), (1,)), ((), ()))


def _dot(a, b):
    return lax.dot_general(a, b, _DN, preferred_element_type=jnp.float32)


def _tc_inv(deg_ref, inv_o):
    inv_o[...] = 1.0 / jnp.maximum(
        deg_ref[0, 0:N, 0:1] + deg_ref[1, 0:N, 0:1], 1.0)


def _tc_r(h_ref, wr_ref, b_ref, r_o):
    r_o[...] = _dot(h_ref[...], wr_ref[...]) + b_ref[...]


def _tc_h(s_ref, inv_ref, r_ref, wl_ref, h_o):
    agg = (s_ref[0, 0:N] + s_ref[1, 0:N]) * inv_ref[...]
    h_o[...] = jnp.maximum(_dot(agg, wl_ref[...]) + r_ref[...], 0.0)


def _tc_out(s_ref, inv_ref, r_ref, wl_ref, w4_ref, b4_ref, out_ref):
    agg = (s_ref[0, 0:N] + s_ref[1, 0:N]) * inv_ref[...]
    h3 = jnp.maximum(_dot(agg, wl_ref[...]) + r_ref[...], 0.0)
    out_ref[...] = (jnp.sum(h3 * w4_ref[...], axis=1, keepdims=True)
                    + b4_ref[0, 0])


def _f32(shape):
    return jax.ShapeDtypeStruct(shape, jnp.float32)


def kernel(x, edge_index, Wl1, Wr1, b1, Wl2, Wr2, b2, Wl3, Wr3, b3, W4, b4):
    src3 = edge_index[0].astype(jnp.int32).reshape(NW, NG, IG, CHUNK)
    dst3 = edge_index[1].astype(jnp.int32).reshape(NW, NG, IG, CHUNK)

    (deg,) = _deg_count(dst3)
    (s1,) = _seg_sum_128(x, src3, dst3)
    r1 = pl.pallas_call(_tc_r, out_shape=_f32((N, 128)))(
        x, Wr1, b1.reshape(1, 128))
    inv = pl.pallas_call(_tc_inv, out_shape=_f32((N, 1)))(deg)

    h1 = pl.pallas_call(_tc_h, out_shape=_f32((N, 128)))(s1, inv, r1, Wl1)
    (s2,) = _seg_sum_128(h1, src3, dst3)
    r2 = pl.pallas_call(_tc_r, out_shape=_f32((N, 128)))(
        h1, Wr2, b2.reshape(1, 128))

    h2 = pl.pallas_call(_tc_h, out_shape=_f32((N, 128)))(s2, inv, r2, Wl2)
    (s3,) = _seg_sum_128(h2, src3, dst3)
    r3 = pl.pallas_call(_tc_r, out_shape=_f32((N, 32)))(
        h2, Wr3, b3.reshape(1, 32))

    out = pl.pallas_call(_tc_out, out_shape=_f32((N, 1)))(
        s3, inv, r3, Wl3, W4, b4.reshape(1, 1))
    return out


# speedup vs baseline: 11.2038x; 1.0056x over previous
import functools

import jax
import jax.numpy as jnp
from jax import lax
from jax.experimental import pallas as pl
from jax.experimental.pallas import tpu as pltpu
from jax.experimental.pallas import tpu_sc as plsc

N = 10000
E = 320000
NC = 2
NS = 16
NW = NC * NS
EPW = E // NW
CHUNK = 100
IG = 25
NG = EPW // (IG * CHUNK)
ACC_ROWS = 10112
RPT = ACC_ROWS // NS
ZC = 96

_MESH = plsc.VectorSubcoreMesh(core_axis_name="c", subcore_axis_name="s")


def _zero_rows(buf, nrows, d):
    def fill(r, carry):
        for k in range(d // 16):
            buf[r, pl.ds(k * 16, 16)] = jnp.zeros((16,), jnp.float32)
        return carry
    lax.fori_loop(0, nrows, fill, 0)


def _zero_acc_slice(acc_s, src, s, zsem):
    zbase = s * RPT
    nfull = RPT // ZC
    rem = RPT - nfull * ZC
    cps = []
    for j in range(nfull):
        cps.append(pltpu.async_copy(
            src.at[pl.ds(0, ZC)], acc_s.at[pl.ds(zbase + j * ZC, ZC)], zsem))
    cps.append(pltpu.async_copy(
        src.at[pl.ds(0, rem)], acc_s.at[pl.ds(zbase + nfull * ZC, rem)], zsem))
    for cp in cps:
        cp.wait()


def _make_seg_sum(d):
    out_type = [jax.ShapeDtypeStruct((NC, ACC_ROWS, d), jnp.float32)]
    scratch = [
        pltpu.VMEM((IG, CHUNK), jnp.int32),
        pltpu.VMEM((IG, CHUNK), jnp.int32),
        pltpu.VMEM((2, CHUNK, d), jnp.float32),
        pltpu.VMEM_SHARED((ACC_ROWS, d), jnp.float32),
        pltpu.SemaphoreType.DMA,
        pltpu.SemaphoreType.DMA,
        pltpu.SemaphoreType.DMA,
        pltpu.SemaphoreType.DMA,
        pltpu.SemaphoreType.DMA,
        pltpu.SemaphoreType.DMA,
    ]

    def body(y_hbm, src_hbm, dst_hbm, out_hbm, src_v, dst_v, rows_v, acc_s,
             gsem0, gsem1, ssem0, ssem1, isem_s, isem_d):
        c = lax.axis_index("c")
        s = lax.axis_index("s")
        w = c * NS + s
        gsems = (gsem0, gsem1)
        ssems = (ssem0, ssem1)

        pltpu.async_copy(src_hbm.at[w, 0], src_v, isem_s)
        pltpu.async_copy(dst_hbm.at[w, 0], dst_v, isem_d)
        _zero_rows(rows_v.at[0], ZC, d)
        _zero_acc_slice(acc_s, rows_v.at[0], s, gsem0)
        plsc.subcore_barrier()

        def gloop(g, carry):
            pltpu.make_async_copy(src_hbm.at[w, g], src_v, isem_s).wait()
            pltpu.make_async_copy(dst_hbm.at[w, g], dst_v, isem_d).wait()
            cpg = [None, None]
            cps = [None, None]
            cpg[0] = pltpu.async_copy(y_hbm.at[src_v.at[0]], rows_v.at[0],
                                      gsems[0])
            for j in range(IG):
                b = j & 1
                if j + 1 < IG:
                    if cps[1 - b] is not None:
                        cps[1 - b].wait()
                    cpg[1 - b] = pltpu.async_copy(
                        y_hbm.at[src_v.at[j + 1]], rows_v.at[1 - b],
                        gsems[1 - b])
                cpg[b].wait()
                cps[b] = pltpu.async_copy(rows_v.at[b], acc_s.at[dst_v.at[j]],
                                          ssems[b], add=True)
            cps[0].wait()
            cps[1].wait()

            @pl.when(g < NG - 1)
            def _prefetch():
                pltpu.async_copy(src_hbm.at[w, g + 1], src_v, isem_s)
                pltpu.async_copy(dst_hbm.at[w, g + 1], dst_v, isem_d)

            return carry

        lax.fori_loop(0, NG, gloop, 0)
        plsc.subcore_barrier()

        sl = pl.ds(s * RPT, RPT)
        pltpu.sync_copy(acc_s.at[sl], out_hbm.at[c, sl])

    return pl.kernel(body, out_type=out_type, mesh=_MESH, scratch_types=scratch)


def _make_deg():
    out_type = [jax.ShapeDtypeStruct((NC, ACC_ROWS, 128), jnp.float32)]
    scratch = [
        pltpu.VMEM((IG, CHUNK), jnp.int32),
        pltpu.VMEM((CHUNK, 128), jnp.float32),
        pltpu.VMEM_SHARED((ACC_ROWS, 128), jnp.float32),
        pltpu.SemaphoreType.DMA,
        pltpu.SemaphoreType.DMA,
    ]

    def body(dst_hbm, deg_hbm, dst_v, ones_v, dacc_s, sem, isem):
        c = lax.axis_index("c")
        s = lax.axis_index("s")
        w = c * NS + s

        pltpu.async_copy(dst_hbm.at[w, 0], dst_v, isem)
        _zero_rows(ones_v, ZC, 128)
        _zero_acc_slice(dacc_s, ones_v, s, sem)

        def fill(r, carry):
            for k in range(8):
                ones_v[r, pl.ds(k * 16, 16)] = jnp.ones((16,), jnp.float32)
            return carry
        lax.fori_loop(0, CHUNK, fill, 0)
        plsc.subcore_barrier()

        def gloop(g, carry):
            pltpu.make_async_copy(dst_hbm.at[w, g], dst_v, isem).wait()
            cps = []
            for j in range(IG):
                cps.append(pltpu.async_copy(ones_v, dacc_s.at[dst_v.at[j]],
                                            sem, add=True))
            for cp in cps:
                cp.wait()

            @pl.when(g < NG - 1)
            def _prefetch():
                pltpu.async_copy(dst_hbm.at[w, g + 1], dst_v, isem)

            return carry

        lax.fori_loop(0, NG, gloop, 0)
        plsc.subcore_barrier()

        sl = pl.ds(s * RPT, RPT)
        pltpu.sync_copy(dacc_s.at[sl], deg_hbm.at[c, sl])

    return pl.kernel(body, out_type=out_type, mesh=_MESH, scratch_types=scratch)


_seg_sum_128 = _make_seg_sum(128)
_deg_count = _make_deg()

_DN = (((1,), (1,)), ((), ()))


def _dot(a, b):
    return lax.dot_general(a, b, _DN, preferred_element_type=jnp.float32)


def _tc_inv(deg_ref, inv_o):
    inv_o[...] = 1.0 / jnp.maximum(
        deg_ref[0, 0:N, 0:1] + deg_ref[1, 0:N, 0:1], 1.0)


def _tc_r(h_ref, wr_ref, b_ref, r_o):
    r_o[...] = _dot(h_ref[...], wr_ref[...]) + b_ref[...]


def _tc_h(s_ref, inv_ref, r_ref, wl_ref, h_o):
    agg = (s_ref[0, 0:N] + s_ref[1, 0:N]) * inv_ref[...]
    h_o[...] = jnp.maximum(_dot(agg, wl_ref[...]) + r_ref[...], 0.0)


def _tc_out(s_ref, inv_ref, r_ref, wl_ref, w4_ref, b4_ref, out_ref):
    agg = (s_ref[0, 0:N] + s_ref[1, 0:N]) * inv_ref[...]
    h3 = jnp.maximum(_dot(agg, wl_ref[...]) + r_ref[...], 0.0)
    out_ref[...] = (jnp.sum(h3 * w4_ref[...], axis=1, keepdims=True)
                    + b4_ref[0, 0])


def _f32(shape):
    return jax.ShapeDtypeStruct(shape, jnp.float32)


def kernel(x, edge_index, Wl1, Wr1, b1, Wl2, Wr2, b2, Wl3, Wr3, b3, W4, b4):
    src3 = edge_index[0].astype(jnp.int32).reshape(NW, NG, IG, CHUNK)
    dst3 = edge_index[1].astype(jnp.int32).reshape(NW, NG, IG, CHUNK)

    (deg,) = _deg_count(dst3)
    (s1,) = _seg_sum_128(x, src3, dst3)
    r1 = pl.pallas_call(_tc_r, out_shape=_f32((N, 128)))(
        x, Wr1, b1.reshape(1, 128))
    inv = pl.pallas_call(_tc_inv, out_shape=_f32((N, 1)))(deg)

    h1 = pl.pallas_call(_tc_h, out_shape=_f32((N, 128)))(s1, inv, r1, Wl1)
    (s2,) = _seg_sum_128(h1, src3, dst3)
    r2 = pl.pallas_call(_tc_r, out_shape=_f32((N, 128)))(
        h1, Wr2, b2.reshape(1, 128))

    h2 = pl.pallas_call(_tc_h, out_shape=_f32((N, 128)))(s2, inv, r2, Wl2)
    (s3,) = _seg_sum_128(h2, src3, dst3)
    r3 = pl.pallas_call(_tc_r, out_shape=_f32((N, 32)))(
        h2, Wr3, b3.reshape(1, 32))

    out = pl.pallas_call(_tc_out, out_shape=_f32((N, 1)))(
        s3, inv, r3, Wl3, W4, b4.reshape(1, 1))
    return out
